# baseline (speedup 1.0000x reference)
from math import ceil

import jax
import jax.numpy as jnp
from jax import lax
from jax.experimental import pallas as pl
from jax.experimental.pallas import tpu as pltpu
from jax.experimental.pallas import tpu_sc as plsc

_K = 17
_N_TARGET = 2000
_NOISE = 0.5

_S = 3456
_L = 16
_NV = _S // _L
_NW = 32
_NG = _S // _L
_SB = 4
_CAP = 64

_F32 = jnp.float32
_I32 = jnp.int32


def _splat(vec, lane):
    idx = jnp.full((_L, 1), lane, dtype=_I32)
    dnums = lax.GatherDimensionNumbers(
        offset_dims=(), collapsed_slice_dims=(0,), start_index_map=(0,))
    return lax.gather(vec, idx, dnums, (1,),
                      mode=lax.GatherScatterMode.PROMISE_IN_BOUNDS)


def _sort2(k, v):
    return plsc.sort_key_val(k, v)


def _rev(x):
    return x[::-1]


def _merge_sorted32_with16(a1, i1, a2, i2, c, ci):
    rc, rci = _rev(c), _rev(ci)
    m = a2 <= rc
    lo = jnp.where(m, a2, rc)
    loi = jnp.where(m, i2, rci)
    ls, lsi = _sort2(lo, loi)
    rl, rli = _rev(ls), _rev(lsi)
    m2 = a1 <= rl
    b1 = jnp.where(m2, a1, rl)
    b1i = jnp.where(m2, i1, rli)
    b2 = jnp.where(m2, rl, a1)
    b2i = jnp.where(m2, rli, i1)
    a1n, i1n = _sort2(b1, b1i)
    a2n, i2n = _sort2(b2, b2i)
    return a1n, i1n, a2n, i2n


def _sqrt_nr(x):
    xc = jnp.maximum(x, _F32(1e-12))
    bits = plsc.bitcast(xc, _I32)
    seed = jnp.full((_L,), 0x5F3759DF, dtype=_I32) - lax.shift_right_logical(
        bits, jnp.full((_L,), 1, dtype=_I32))
    y = plsc.bitcast(seed, _F32)
    half = _F32(0.5) * xc
    for _ in range(3):
        y = y * (_F32(1.5) - half * y * y)
    return xc * y


def _knn_sc_body(xs_h, ys_h, zs_h, vtop_h, itop_h,
                 xs_v, ys_v, zs_v, dbuf, cval, cidx, obv, obi):
    cc = lax.axis_index("c")
    ss = lax.axis_index("s")
    wid = ss * 2 + cc

    pltpu.sync_copy(xs_h, xs_v)
    pltpu.sync_copy(ys_h, ys_v)
    pltpu.sync_copy(zs_h, zs_v)

    inf16 = jnp.full((_L,), jnp.inf, dtype=_F32)
    ones16 = jnp.full((_L,), 1, dtype=_I32)
    iota16 = lax.iota(_I32, _L)

    ng = (_NG - wid + _NW - 1) // _NW

    def group_body(gi, carry):
        g = wid + gi * _NW
        base = g * _L
        qx16 = xs_v[pl.ds(base, _L)]
        qy16 = ys_v[pl.ds(base, _L)]
        qz16 = zs_v[pl.ds(base, _L)]

        for sb in range(_L // _SB):
            qs = [sb * _SB + l for l in range(_SB)]
            qxs = [_splat(qx16, q) for q in qs]
            qys = [_splat(qy16, q) for q in qs]
            qzs = [_splat(qz16, q) for q in qs]

            def p1_body(j, c):
                lmins, m2s = c
                off = j * _L
                xv = xs_v[pl.ds(off, _L)]
                yv = ys_v[pl.ds(off, _L)]
                zv = zs_v[pl.ds(off, _L)]
                nl, n2 = [], []
                for l in range(_SB):
                    dx = xv - qxs[l]
                    dy = yv - qys[l]
                    dz = zv - qzs[l]
                    d2 = (dx * dx + dy * dy) + dz * dz
                    dbuf[qs[l], pl.ds(off, _L)] = d2
                    t = jnp.minimum(lmins[l], d2)
                    n2.append(jnp.minimum(m2s[l], jnp.maximum(lmins[l], d2)))
                    nl.append(t)
                return (tuple(nl), tuple(n2))

            lmins, m2s = lax.fori_loop(
                0, _NV, p1_body,
                (tuple(inf16 for _ in range(_SB)),
                 tuple(inf16 for _ in range(_SB))))

            taus = []
            for l in range(_SB):
                a = jnp.sort(lmins[l])
                b = jnp.sort(m2s[l])
                hi = jnp.maximum(a, _rev(b))
                hs = jnp.sort(hi)
                taus.append(_splat(hs, 1))

            for l in range(_SB):
                for blk in range(_CAP // _L):
                    cval[qs[l], pl.ds(blk * _L, _L)] = inf16

            def p2_body(j, c):
                cnts, colv = c
                off = j * _L
                ncnts = []
                for l in range(_SB):
                    v = dbuf[qs[l], pl.ds(off, _L)]
                    msk = v <= taus[l]
                    pos = cnts[l] + plsc.cumsum(ones16, mask=msk)
                    plsc.store_scatter(cval.at[qs[l]], [pos], v, mask=msk)
                    plsc.store_scatter(cidx.at[qs[l]], [pos], colv, mask=msk)
                    nc = cnts[l] + plsc.all_reduce_population_count(msk)
                    ncnts.append(jnp.minimum(nc, jnp.full((_L,), _CAP - _L - 1,
                                                          dtype=_I32)))
                return (tuple(ncnts), colv + _L)

            m1 = jnp.full((_L,), -1, dtype=_I32)
            lax.fori_loop(0, _NV, p2_body,
                          (tuple(m1 for _ in range(_SB)), iota16))

            for l in range(_SB):
                row = jnp.full((_L,), base + qs[l], dtype=_I32)
                vb, ib = [], []
                for blk in range(_CAP // _L):
                    v = cval[qs[l], pl.ds(blk * _L, _L)]
                    i = cidx[qs[l], pl.ds(blk * _L, _L)]
                    vb.append(jnp.where(i == row, inf16, v))
                    ib.append(i)
                k0, i0 = _sort2(vb[0], ib[0])
                k1, i1 = _sort2(vb[1], ib[1])
                rk, rki = _rev(k1), _rev(i1)
                m = k0 <= rk
                lo = jnp.where(m, k0, rk)
                loi = jnp.where(m, i0, rki)
                hi = jnp.where(m, rk, k0)
                hii = jnp.where(m, rki, i0)
                a1, i1s = _sort2(lo, loi)
                a2, i2s = _sort2(hi, hii)
                for blk in range(2, _CAP // _L):
                    cs, cis = _sort2(vb[blk], ib[blk])
                    a1, i1s, a2, i2s = _merge_sorted32_with16(
                        a1, i1s, a2, i2s, cs, cis)
                obv[qs[l], pl.ds(0, _L)] = _sqrt_nr(a1)
                obv[qs[l], pl.ds(_L, _L)] = _sqrt_nr(a2)
                obi[qs[l], pl.ds(0, _L)] = i1s
                obi[qs[l], pl.ds(_L, _L)] = i2s

        pltpu.sync_copy(obv, vtop_h.at[pl.ds(base, _L)])
        pltpu.sync_copy(obi, itop_h.at[pl.ds(base, _L)])
        return carry

    lax.fori_loop(0, ng, group_body, 0)


def _knn_sparsecore(xs, ys, zs):
    mesh = plsc.VectorSubcoreMesh(core_axis_name="c", subcore_axis_name="s",
                                  num_cores=2, num_subcores=16)
    f = pl.kernel(
        _knn_sc_body,
        out_type=[
            jax.ShapeDtypeStruct((_S, 2 * _L), jnp.float32),
            jax.ShapeDtypeStruct((_S, 2 * _L), jnp.int32),
        ],
        mesh=mesh,
        scratch_types=[
            pltpu.VMEM((_S,), jnp.float32),
            pltpu.VMEM((_S,), jnp.float32),
            pltpu.VMEM((_S,), jnp.float32),
            pltpu.VMEM((_L, _S), jnp.float32),
            pltpu.VMEM((_L, _CAP), jnp.float32),
            pltpu.VMEM((_L, _CAP), jnp.int32),
            pltpu.VMEM((_L, 2 * _L), jnp.float32),
            pltpu.VMEM((_L, 2 * _L), jnp.int32),
        ],
        compiler_params=pltpu.CompilerParams(needs_layout_passes=False),
    )
    return f(xs, ys, zs)


def kernel(positions, cell, numbers, noise_eps):
    frac = positions @ jnp.linalg.inv(cell)
    n = positions.shape[0]
    replicates = ceil((_N_TARGET / n) ** (1.0 / 3.0))
    r = replicates
    ax = jnp.arange(r, dtype=frac.dtype)
    offs = jnp.stack(jnp.meshgrid(ax, ax, ax, indexing="ij"), axis=-1).reshape(-1, 3)
    sc = (frac[None, :, :] + offs[:, None, :]).reshape(-1, 3)
    sc = sc + _NOISE * noise_eps
    cart = sc @ cell
    S = cart.shape[0]

    xs = cart[:, 0]
    ys = cart[:, 1]
    zs = cart[:, 2]

    vtop, itop = _knn_sparsecore(xs, ys, zs)

    dists = vtop[:, :_K]
    idx = itop[:, :_K]
    src = jnp.repeat(jnp.arange(S, dtype=jnp.int32), _K)
    dst = idx.reshape(-1)
    numbers_rep = jnp.tile(numbers, r ** 3)
    return dists, src, dst, numbers_rep, jnp.float32(_NOISE)


# speedup vs baseline: 1.0821x; 1.0821x over previous
from math import ceil

import jax
import jax.numpy as jnp
from jax import lax
from jax.experimental import pallas as pl
from jax.experimental.pallas import tpu as pltpu
from jax.experimental.pallas import tpu_sc as plsc

_K = 17
_N_TARGET = 2000
_NOISE = 0.5

_S = 3456
_L = 16
_NV = _S // _L
_NW = 32
_SB = 4
_NSB = _S // _SB // _NW
_CAP = 64
_JCAP = 48

_F32 = jnp.float32
_I32 = jnp.int32


def _splat(vec, lane):
    if isinstance(lane, int):
        idx = jnp.full((_L, 1), lane, dtype=_I32)
    else:
        idx = lax.broadcast_in_dim(lane.astype(_I32), (_L, 1), ())
    dnums = lax.GatherDimensionNumbers(
        offset_dims=(), collapsed_slice_dims=(0,), start_index_map=(0,))
    return lax.gather(vec, idx, dnums, (1,),
                      mode=lax.GatherScatterMode.PROMISE_IN_BOUNDS)


def _sort2(k, v):
    return plsc.sort_key_val(k, v)


def _rev(x):
    return x[::-1]


def _merge_sorted32_with16(a1, i1, a2, i2, c, ci):
    rc, rci = _rev(c), _rev(ci)
    m = a2 <= rc
    lo = jnp.where(m, a2, rc)
    loi = jnp.where(m, i2, rci)
    ls, lsi = _sort2(lo, loi)
    rl, rli = _rev(ls), _rev(lsi)
    m2 = a1 <= rl
    b1 = jnp.where(m2, a1, rl)
    b1i = jnp.where(m2, i1, rli)
    b2 = jnp.where(m2, rl, a1)
    b2i = jnp.where(m2, rli, i1)
    a1n, i1n = _sort2(b1, b1i)
    a2n, i2n = _sort2(b2, b2i)
    return a1n, i1n, a2n, i2n


def _sqrt_nr(x):
    xc = jnp.maximum(x, _F32(1e-12))
    bits = plsc.bitcast(xc, _I32)
    seed = jnp.full((_L,), 0x5F3759DF, dtype=_I32) - lax.shift_right_logical(
        bits, jnp.full((_L,), 1, dtype=_I32))
    y = plsc.bitcast(seed, _F32)
    half = _F32(0.5) * xc
    for _ in range(3):
        y = y * (_F32(1.5) - half * y * y)
    return xc * y


def _knn_sc_body(xs_h, ys_h, zs_h, vtop_h, itop_h,
                 xs_v, ys_v, zs_v, db0, db1, db2, db3,
                 jl0, jl1, jl2, jl3, ci0, ci1, ci2, ci3, obv, obi):
    dbufs = [db0, db1, db2, db3]
    jlsts = [jl0, jl1, jl2, jl3]
    cidxs = [ci0, ci1, ci2, ci3]
    cc = lax.axis_index("c")
    ss = lax.axis_index("s")
    wid = ss * 2 + cc

    pltpu.sync_copy(xs_h, xs_v)
    pltpu.sync_copy(ys_h, ys_v)
    pltpu.sync_copy(zs_h, zs_v)

    inf16 = jnp.full((_L,), jnp.inf, dtype=_F32)
    ones16 = jnp.full((_L,), 1, dtype=_I32)
    iota16 = lax.iota(_I32, _L)
    lane0 = iota16 == 0
    sent16 = jnp.full((_L,), _NV, dtype=_I32)

    for q in range(_SB):
        dbufs[q][pl.ds(_S, _L)] = inf16

    def sb_body(t, carry):
        sbg = wid + t * _NW
        base = sbg * _SB
        grp = base - lax.rem(base, _L)
        off_in = base - grp
        qx16 = xs_v[pl.ds(grp, _L)]
        qy16 = ys_v[pl.ds(grp, _L)]
        qz16 = zs_v[pl.ds(grp, _L)]
        qxs = [_splat(qx16, off_in + l) for l in range(_SB)]
        qys = [_splat(qy16, off_in + l) for l in range(_SB)]
        qzs = [_splat(qz16, off_in + l) for l in range(_SB)]

        def p1_body(j, c):
            lmins, m2s = c
            off = j * _L
            xv = xs_v[pl.ds(off, _L)]
            yv = ys_v[pl.ds(off, _L)]
            zv = zs_v[pl.ds(off, _L)]
            nl, n2 = [], []
            for l in range(_SB):
                dx = xv - qxs[l]
                dy = yv - qys[l]
                dz = zv - qzs[l]
                d2 = (dx * dx + dy * dy) + dz * dz
                dbufs[l][pl.ds(off, _L)] = d2
                nt = jnp.minimum(lmins[l], d2)
                n2.append(jnp.minimum(m2s[l], jnp.maximum(lmins[l], d2)))
                nl.append(nt)
            return (tuple(nl), tuple(n2))

        lmins, m2s = lax.fori_loop(
            0, _NV, p1_body,
            (tuple(inf16 for _ in range(_SB)),
             tuple(inf16 for _ in range(_SB))), unroll=2)

        taus = []
        for l in range(_SB):
            a = jnp.sort(lmins[l])
            b = jnp.sort(m2s[l])
            hi = jnp.maximum(a, _rev(b))
            hs = jnp.sort(hi)
            taus.append(_splat(hs, 1))

        for l in range(_SB):
            for blk in range(_JCAP // _L):
                jlsts[l][pl.ds(blk * _L, _L)] = sent16

        def p2a_body(j, c):
            cnts, jv = c
            off = j * _L
            ncnts = []
            for l in range(_SB):
                v = dbufs[l][pl.ds(off, _L)]
                msk = v <= taus[l]
                pc = plsc.all_reduce_population_count(msk)
                wm = (pc > 0) & lane0
                plsc.store_scatter(jlsts[l], [cnts[l]], jv, mask=wm)
                inc = jnp.minimum(pc, ones16)
                ncnts.append(jnp.minimum(
                    cnts[l] + inc, jnp.full((_L,), _JCAP - 1, dtype=_I32)))
            return (tuple(ncnts), jv + 1)

        z16 = jnp.zeros((_L,), dtype=_I32)
        jcnts, _ = lax.fori_loop(
            0, _NV, p2a_body,
            (tuple(z16 for _ in range(_SB)), z16), unroll=2)

        njs = [jnp.max(jcnts[l]) for l in range(_SB)]
        njmax = jnp.maximum(jnp.maximum(njs[0], njs[1]),
                            jnp.maximum(njs[2], njs[3]))

        for l in range(_SB):
            for blk in range(_CAP // _L):
                cidxs[l][pl.ds(blk * _L, _L)] = jnp.full((_L,), _S, dtype=_I32)

        def p2b_body(i, c):
            cnts = c
            ii = lax.broadcast_in_dim(i, (_L,), ())
            ncnts = []
            for l in range(_SB):
                jspl = plsc.load_gather(jlsts[l], [ii])
                addr = jspl * _L + iota16
                v = plsc.load_gather(dbufs[l], [addr])
                msk = v <= taus[l]
                pos = cnts[l] + plsc.cumsum(ones16, mask=msk)
                plsc.store_scatter(cidxs[l], [pos], addr, mask=msk)
                nc = cnts[l] + plsc.all_reduce_population_count(msk)
                ncnts.append(jnp.minimum(
                    nc, jnp.full((_L,), _CAP - _L - 1, dtype=_I32)))
            return tuple(ncnts)

        m1 = jnp.full((_L,), -1, dtype=_I32)
        lax.fori_loop(0, njmax, p2b_body, tuple(m1 for _ in range(_SB)))

        for l in range(_SB):
            row = lax.broadcast_in_dim(base + l, (_L,), ()).astype(_I32)
            vb, ib = [], []
            for blk in range(_CAP // _L):
                i = cidxs[l][pl.ds(blk * _L, _L)]
                v = plsc.load_gather(dbufs[l], [i])
                vb.append(jnp.where(i == row, inf16, v))
                ib.append(i)
            k0, i0 = _sort2(vb[0], ib[0])
            k1, i1 = _sort2(vb[1], ib[1])
            rk, rki = _rev(k1), _rev(i1)
            m = k0 <= rk
            lo = jnp.where(m, k0, rk)
            loi = jnp.where(m, i0, rki)
            hi = jnp.where(m, rk, k0)
            hii = jnp.where(m, rki, i0)
            a1, i1s = _sort2(lo, loi)
            a2, i2s = _sort2(hi, hii)
            for blk in range(2, _CAP // _L):
                cs, cis = _sort2(vb[blk], ib[blk])
                a1, i1s, a2, i2s = _merge_sorted32_with16(
                    a1, i1s, a2, i2s, cs, cis)
            obv[l, pl.ds(0, _L)] = _sqrt_nr(a1)
            obv[l, pl.ds(_L, _L)] = _sqrt_nr(a2)
            obi[l, pl.ds(0, _L)] = i1s
            obi[l, pl.ds(_L, _L)] = i2s

        pltpu.sync_copy(obv, vtop_h.at[pl.ds(base, _SB)])
        pltpu.sync_copy(obi, itop_h.at[pl.ds(base, _SB)])
        return carry

    lax.fori_loop(0, _NSB, sb_body, 0)


def _knn_sparsecore(xs, ys, zs):
    mesh = plsc.VectorSubcoreMesh(core_axis_name="c", subcore_axis_name="s",
                                  num_cores=2, num_subcores=16)
    f = pl.kernel(
        _knn_sc_body,
        out_type=[
            jax.ShapeDtypeStruct((_S, 2 * _L), jnp.float32),
            jax.ShapeDtypeStruct((_S, 2 * _L), jnp.int32),
        ],
        mesh=mesh,
        scratch_types=[
            pltpu.VMEM((_S,), jnp.float32),
            pltpu.VMEM((_S,), jnp.float32),
            pltpu.VMEM((_S,), jnp.float32),
        ] + [pltpu.VMEM((_S + _L,), jnp.float32) for _ in range(_SB)]
          + [pltpu.VMEM((_JCAP,), jnp.int32) for _ in range(_SB)]
          + [pltpu.VMEM((_CAP,), jnp.int32) for _ in range(_SB)] + [
            pltpu.VMEM((_SB, 2 * _L), jnp.float32),
            pltpu.VMEM((_SB, 2 * _L), jnp.int32),
        ],
        compiler_params=pltpu.CompilerParams(needs_layout_passes=False),
    )
    return f(xs, ys, zs)


def kernel(positions, cell, numbers, noise_eps):
    frac = positions @ jnp.linalg.inv(cell)
    n = positions.shape[0]
    replicates = ceil((_N_TARGET / n) ** (1.0 / 3.0))
    r = replicates
    ax = jnp.arange(r, dtype=frac.dtype)
    offs = jnp.stack(jnp.meshgrid(ax, ax, ax, indexing="ij"), axis=-1).reshape(-1, 3)
    sc = (frac[None, :, :] + offs[:, None, :]).reshape(-1, 3)
    sc = sc + _NOISE * noise_eps
    cart = sc @ cell
    S = cart.shape[0]

    xs = cart[:, 0]
    ys = cart[:, 1]
    zs = cart[:, 2]

    vtop, itop = _knn_sparsecore(xs, ys, zs)

    dists = vtop[:, :_K]
    idx = itop[:, :_K]
    src = jnp.repeat(jnp.arange(S, dtype=jnp.int32), _K)
    dst = idx.reshape(-1)
    numbers_rep = jnp.tile(numbers, r ** 3)
    return dists, src, dst, numbers_rep, jnp.float32(_NOISE)


# speedup vs baseline: 1.8324x; 1.6933x over previous
from math import ceil

import jax
import jax.numpy as jnp
from jax import lax
from jax.experimental import pallas as pl
from jax.experimental.pallas import tpu as pltpu
from jax.experimental.pallas import tpu_sc as plsc

_K = 17
_N_TARGET = 2000
_NOISE = 0.5

_S = 3456
_L = 16
_NV = _S // _L
_NW = 32
_SB = 4
_NSB = _S // _SB // _NW
_CAP = 64
_JCAP = 48

_F32 = jnp.float32
_I32 = jnp.int32


def _splat(vec, lane):
    if isinstance(lane, int):
        idx = jnp.full((_L, 1), lane, dtype=_I32)
    else:
        idx = lax.broadcast_in_dim(lane.astype(_I32), (_L, 1), ())
    dnums = lax.GatherDimensionNumbers(
        offset_dims=(), collapsed_slice_dims=(0,), start_index_map=(0,))
    return lax.gather(vec, idx, dnums, (1,),
                      mode=lax.GatherScatterMode.PROMISE_IN_BOUNDS)


def _sort2(k, v):
    return plsc.sort_key_val(k, v)


def _rev(x):
    return x[::-1]


def _merge_sorted32_with16(a1, i1, a2, i2, c, ci):
    rc, rci = _rev(c), _rev(ci)
    m = a2 <= rc
    lo = jnp.where(m, a2, rc)
    loi = jnp.where(m, i2, rci)
    ls, lsi = _sort2(lo, loi)
    rl, rli = _rev(ls), _rev(lsi)
    m2 = a1 <= rl
    b1 = jnp.where(m2, a1, rl)
    b1i = jnp.where(m2, i1, rli)
    b2 = jnp.where(m2, rl, a1)
    b2i = jnp.where(m2, rli, i1)
    a1n, i1n = _sort2(b1, b1i)
    a2n, i2n = _sort2(b2, b2i)
    return a1n, i1n, a2n, i2n


def _sqrt_nr(x):
    xc = jnp.maximum(x, _F32(1e-12))
    bits = plsc.bitcast(xc, _I32)
    seed = jnp.full((_L,), 0x5F3759DF, dtype=_I32) - lax.shift_right_logical(
        bits, jnp.full((_L,), 1, dtype=_I32))
    y = plsc.bitcast(seed, _F32)
    half = _F32(0.5) * xc
    for _ in range(3):
        y = y * (_F32(1.5) - half * y * y)
    return xc * y


def _knn_sc_body(xs_h, ys_h, zs_h, vtop_h, itop_h,
                 xs_v, ys_v, zs_v, db0, db1, db2, db3,
                 ubuf, jlstu, ci0, ci1, ci2, ci3, obv, obi):
    dbufs = [db0, db1, db2, db3]
    cidxs = [ci0, ci1, ci2, ci3]
    cc = lax.axis_index("c")
    ss = lax.axis_index("s")
    wid = ss * 2 + cc

    pltpu.sync_copy(xs_h, xs_v)
    pltpu.sync_copy(ys_h, ys_v)
    pltpu.sync_copy(zs_h, zs_v)

    inf16 = jnp.full((_L,), jnp.inf, dtype=_F32)
    ones16 = jnp.full((_L,), 1, dtype=_I32)
    iota16 = lax.iota(_I32, _L)
    lane0 = iota16 == 0
    sent16 = jnp.full((_L,), _NV, dtype=_I32)

    for q in range(_SB):
        dbufs[q][pl.ds(_S, _L)] = inf16

    def sb_body(t, carry):
        sbg = wid + t * _NW
        base = sbg * _SB
        grp = base - lax.rem(base, _L)
        off_in = base - grp
        qx16 = xs_v[pl.ds(grp, _L)]
        qy16 = ys_v[pl.ds(grp, _L)]
        qz16 = zs_v[pl.ds(grp, _L)]
        qxs = [_splat(qx16, off_in + l) for l in range(_SB)]
        qys = [_splat(qy16, off_in + l) for l in range(_SB)]
        qzs = [_splat(qz16, off_in + l) for l in range(_SB)]

        def p1_body(j, c):
            lmins, m2s = c
            off = j * _L
            xv = xs_v[pl.ds(off, _L)]
            yv = ys_v[pl.ds(off, _L)]
            zv = zs_v[pl.ds(off, _L)]
            nl, n2, d2s = [], [], []
            for l in range(_SB):
                dx = xv - qxs[l]
                dy = yv - qys[l]
                dz = zv - qzs[l]
                d2 = (dx * dx + dy * dy) + dz * dz
                dbufs[l][pl.ds(off, _L)] = d2
                d2s.append(d2)
                nt = jnp.minimum(lmins[l], d2)
                n2.append(jnp.minimum(m2s[l], jnp.maximum(lmins[l], d2)))
                nl.append(nt)
            um = jnp.minimum(jnp.minimum(d2s[0], d2s[1]),
                             jnp.minimum(d2s[2], d2s[3]))
            ubuf[pl.ds(off, _L)] = um
            return (tuple(nl), tuple(n2))

        lmins, m2s = lax.fori_loop(
            0, _NV, p1_body,
            (tuple(inf16 for _ in range(_SB)),
             tuple(inf16 for _ in range(_SB))))

        taus = []
        for l in range(_SB):
            a = jnp.sort(lmins[l])
            b = jnp.sort(m2s[l])
            hi = jnp.maximum(a, _rev(b))
            hs = jnp.sort(hi)
            taus.append(_splat(hs, 1))

        taumax = jnp.maximum(jnp.maximum(taus[0], taus[1]),
                             jnp.maximum(taus[2], taus[3]))
        for blk in range(_JCAP // _L):
            jlstu[pl.ds(blk * _L, _L)] = sent16

        def p2a_body(j, c):
            cntu, jv = c
            u = ubuf[pl.ds(j * _L, _L)]
            msk = u <= taumax
            pc = plsc.all_reduce_population_count(msk)
            wm = (pc > 0) & lane0
            plsc.store_scatter(jlstu, [cntu], jv, mask=wm)
            inc = jnp.minimum(pc, ones16)
            ncnt = jnp.minimum(cntu + inc,
                               jnp.full((_L,), _JCAP - 1, dtype=_I32))
            return (ncnt, jv + 1)

        z16 = jnp.zeros((_L,), dtype=_I32)
        cntu, _ = lax.fori_loop(0, _NV, p2a_body, (z16, z16))
        njmax = jnp.max(cntu)
        jv0 = jlstu[pl.ds(0, _L)]
        jv1 = jlstu[pl.ds(_L, _L)]
        jv2 = jlstu[pl.ds(2 * _L, _L)]

        for l in range(_SB):
            for blk in range(_CAP // _L):
                cidxs[l][pl.ds(blk * _L, _L)] = jnp.full((_L,), _S, dtype=_I32)

        def p2b_body(i, c):
            cnts = c
            lane = lax.rem(i, _L)
            jsel = jnp.where(i < _L, jv0, jnp.where(i < 2 * _L, jv1, jv2))
            jspl = _splat(jsel, lane)
            addr = jspl * _L + iota16
            ncnts = []
            for l in range(_SB):
                v = plsc.load_gather(dbufs[l], [addr])
                msk = v <= taus[l]
                pos = cnts[l] + plsc.cumsum(ones16, mask=msk)
                plsc.store_scatter(cidxs[l], [pos], addr, mask=msk)
                nc = cnts[l] + plsc.all_reduce_population_count(msk)
                ncnts.append(jnp.minimum(
                    nc, jnp.full((_L,), _CAP - _L - 1, dtype=_I32)))
            return tuple(ncnts)

        m1 = jnp.full((_L,), -1, dtype=_I32)
        lax.fori_loop(0, njmax, p2b_body, tuple(m1 for _ in range(_SB)))

        for l in range(_SB):
            row = lax.broadcast_in_dim(base + l, (_L,), ()).astype(_I32)
            vb, ib = [], []
            for blk in range(_CAP // _L):
                i = cidxs[l][pl.ds(blk * _L, _L)]
                v = plsc.load_gather(dbufs[l], [i])
                vb.append(jnp.where(i == row, inf16, v))
                ib.append(i)
            k0, i0 = _sort2(vb[0], ib[0])
            k1, i1 = _sort2(vb[1], ib[1])
            rk, rki = _rev(k1), _rev(i1)
            m = k0 <= rk
            lo = jnp.where(m, k0, rk)
            loi = jnp.where(m, i0, rki)
            hi = jnp.where(m, rk, k0)
            hii = jnp.where(m, rki, i0)
            a1, i1s = _sort2(lo, loi)
            a2, i2s = _sort2(hi, hii)
            for blk in range(2, _CAP // _L):
                cs, cis = _sort2(vb[blk], ib[blk])
                a1, i1s, a2, i2s = _merge_sorted32_with16(
                    a1, i1s, a2, i2s, cs, cis)
            obv[l, pl.ds(0, _L)] = _sqrt_nr(a1)
            obv[l, pl.ds(_L, _L)] = _sqrt_nr(a2)
            obi[l, pl.ds(0, _L)] = i1s
            obi[l, pl.ds(_L, _L)] = i2s

        pltpu.sync_copy(obv, vtop_h.at[pl.ds(base, _SB)])
        pltpu.sync_copy(obi, itop_h.at[pl.ds(base, _SB)])
        return carry

    lax.fori_loop(0, _NSB, sb_body, 0)


def _knn_sparsecore(xs, ys, zs):
    mesh = plsc.VectorSubcoreMesh(core_axis_name="c", subcore_axis_name="s",
                                  num_cores=2, num_subcores=16)
    f = pl.kernel(
        _knn_sc_body,
        out_type=[
            jax.ShapeDtypeStruct((_S, 2 * _L), jnp.float32),
            jax.ShapeDtypeStruct((_S, 2 * _L), jnp.int32),
        ],
        mesh=mesh,
        scratch_types=[
            pltpu.VMEM((_S,), jnp.float32),
            pltpu.VMEM((_S,), jnp.float32),
            pltpu.VMEM((_S,), jnp.float32),
        ] + [pltpu.VMEM((_S + _L,), jnp.float32) for _ in range(_SB)]
          + [pltpu.VMEM((_S,), jnp.float32), pltpu.VMEM((_JCAP,), jnp.int32)]
          + [pltpu.VMEM((_CAP,), jnp.int32) for _ in range(_SB)] + [
            pltpu.VMEM((_SB, 2 * _L), jnp.float32),
            pltpu.VMEM((_SB, 2 * _L), jnp.int32),
        ],
        compiler_params=pltpu.CompilerParams(needs_layout_passes=False),
    )
    return f(xs, ys, zs)


def kernel(positions, cell, numbers, noise_eps):
    frac = positions @ jnp.linalg.inv(cell)
    n = positions.shape[0]
    replicates = ceil((_N_TARGET / n) ** (1.0 / 3.0))
    r = replicates
    ax = jnp.arange(r, dtype=frac.dtype)
    offs = jnp.stack(jnp.meshgrid(ax, ax, ax, indexing="ij"), axis=-1).reshape(-1, 3)
    sc = (frac[None, :, :] + offs[:, None, :]).reshape(-1, 3)
    sc = sc + _NOISE * noise_eps
    cart = sc @ cell
    S = cart.shape[0]

    xs = cart[:, 0]
    ys = cart[:, 1]
    zs = cart[:, 2]

    vtop, itop = _knn_sparsecore(xs, ys, zs)

    dists = vtop[:, :_K]
    idx = itop[:, :_K]
    src = jnp.repeat(jnp.arange(S, dtype=jnp.int32), _K)
    dst = idx.reshape(-1)
    numbers_rep = jnp.tile(numbers, r ** 3)
    return dists, src, dst, numbers_rep, jnp.float32(_NOISE)
